# baseline (speedup 1.0000x reference)
import jax
import jax.numpy as jnp
import numpy as np
from jax.experimental import pallas as pl
from jax.experimental.pallas import tpu as pltpu
from jax.experimental.pallas import tpu_sc as plsc

_B = 2048
_W = 128
_G = 4
_SCV = 16


def _sc_gather(nodes2, idx, ep):
    feat = nodes2.shape[1]
    idx2 = idx.reshape(1, ep)
    mesh = plsc.VectorSubcoreMesh(core_axis_name="core", subcore_axis_name="subcore")

    @pl.kernel(out_type=jax.ShapeDtypeStruct((ep, feat), nodes2.dtype),
               mesh=mesh)
    def gather_kernel(x_hbm, i_hbm, o_hbm):
        def body(i_vmem, o_vmem):
            pltpu.sync_copy(x_hbm.at[i_vmem.at[0]], o_vmem)

        pltpu.emit_pipeline(
            body,
            grid=(ep // _W,),
            in_specs=[pl.BlockSpec((1, _W), index_map=lambda i: (0, i))],
            out_specs=[pl.BlockSpec((_W, feat), index_map=lambda i: (i, 0))],
            core_axis_name=("core", "subcore"),
            dimension_semantics=(pltpu.PARALLEL,),
        )(i_hbm, o_hbm)

    return gather_kernel(nodes2, idx2)


def _regroup(kbig, g, num_rel, out_f):
    wide = kbig.shape[0]
    s = num_rel // g
    return (kbig.reshape(wide, g, s * out_f)
            .transpose(1, 0, 2)
            .reshape(g * wide, s * out_f))


def _tc_messages(xa, xb, send3, type3, kgrp, e, ep, in_f, rf, out_f):
    nb = ep // _B
    nh = nb // 2
    wide = 2 * in_f
    s = rf // out_f // _G
    yw = s * out_f

    def body(xa_ref, xb_ref, s_ref, t_ref, k_ref, o_ref):
        half = pl.program_id(0) >= nh
        xw = jnp.where(half, xb_ref[...], xa_ref[...])
        sv = s_ref[0]
        tv = t_ref[0]
        codev = (tv << 1) | (sv & 1)
        code = codev.reshape(_B, 1)
        slot = ((code >> 1) // s) * 2 + (code & 1)
        jrel = (code >> 1) % s
        xb = xw.astype(jnp.bfloat16)
        xg = jnp.concatenate([xb] * _G, axis=1)
        gcol = jax.lax.broadcasted_iota(jnp.int32, (_B, wide * _G), 1)
        xg = jnp.where((gcol // in_f) == slot, xg, jnp.bfloat16(0))
        y = jnp.dot(xg, k_ref[...], preferred_element_type=jnp.float32)
        jcol = jax.lax.broadcasted_iota(jnp.int32, (_B, yw), 1)
        ym = jnp.where((jcol // out_f) == jrel, y, 0.0)
        acc = ym[:, 0:128]
        for c in range(1, yw // 128):
            acc = acc + ym[:, 128 * c:128 * (c + 1)]
        res = acc[:, 0:out_f]
        for j in range(1, 128 // out_f):
            res = res + acc[:, out_f * j:out_f * (j + 1)]
        o_ref[...] = jnp.swapaxes(res, 0, 1)

    return pl.pallas_call(
        body,
        grid=(nb,),
        in_specs=[
            pl.BlockSpec((_B, wide), lambda i: (jnp.minimum(i, nh - 1), 0)),
            pl.BlockSpec((_B, wide), lambda i: (jnp.maximum(i - nh, 0), 0)),
            pl.BlockSpec((1, 1, _B), lambda i: (i, 0, 0)),
            pl.BlockSpec((1, 1, _B), lambda i: (i, 0, 0)),
            pl.BlockSpec((_G * wide, yw), lambda i: (0, 0)),
        ],
        out_specs=pl.BlockSpec((out_f, _B), lambda i: (0, i)),
        out_shape=jax.ShapeDtypeStruct((out_f, e), jnp.float32),
    )(xa, xb, send3, type3, kgrp)


def kernel(nodes, senders, edge_types, kernels):
    e = senders.shape[0]
    num_rel, in_f, out_f = kernels.shape
    rf = num_rel * out_f
    nodes2 = nodes.reshape(nodes.shape[0] // 2, 2 * in_f)

    lcm = int(np.lcm(_B, _W))
    ep = ((e + lcm - 1) // lcm) * lcm
    pad = ep - e
    sp = jnp.pad(senders, (0, pad))
    tp = jnp.pad(edge_types, (0, pad))
    nb = ep // _B

    ep2 = ep // 2
    idxh = sp >> 1
    xa = _sc_gather(nodes2, idxh[:ep2], ep2)
    xb = _sc_gather(nodes2, idxh[ep2:], ep2)

    kflat = jnp.transpose(kernels, (1, 0, 2)).reshape(in_f, rf)
    kbig = jnp.concatenate([kflat, kflat], axis=0).astype(jnp.bfloat16)
    kgrp = _regroup(kbig, _G, num_rel, out_f)

    out = _tc_messages(xa, xb, sp.reshape(nb, 1, _B), tp.reshape(nb, 1, _B),
                       kgrp, e, ep, in_f, rf, out_f)
    return out.T


# speedup vs baseline: 1.2307x; 1.2307x over previous
import jax
import jax.numpy as jnp
import numpy as np
from jax.experimental import pallas as pl
from jax.experimental.pallas import tpu as pltpu
from jax.experimental.pallas import tpu_sc as plsc

_B = 2048
_W = 128
_G = 4
_SCV = 16


def _sc_gather(nodes2, idx, ep):
    feat = nodes2.shape[1]
    idx2 = idx.reshape(1, ep)
    mesh = plsc.VectorSubcoreMesh(core_axis_name="core", subcore_axis_name="subcore")

    @pl.kernel(out_type=jax.ShapeDtypeStruct((ep, feat), nodes2.dtype),
               mesh=mesh)
    def gather_kernel(x_hbm, i_hbm, o_hbm):
        def body(i_vmem, o_vmem):
            pltpu.sync_copy(x_hbm.at[i_vmem.at[0]], o_vmem)

        pltpu.emit_pipeline(
            body,
            grid=(ep // _W,),
            in_specs=[pl.BlockSpec((1, _W), index_map=lambda i: (0, i))],
            out_specs=[pl.BlockSpec((_W, feat), index_map=lambda i: (i, 0))],
            core_axis_name=("core", "subcore"),
            dimension_semantics=(pltpu.PARALLEL,),
        )(i_hbm, o_hbm)

    return gather_kernel(nodes2, idx2)


def _regroup(kbig, g, num_rel, out_f):
    wide = kbig.shape[0]
    s = num_rel // g
    return (kbig.reshape(wide, g, s * out_f)
            .transpose(1, 0, 2)
            .reshape(g * wide, s * out_f))


def _tc_messages(x128, send3, type3, kgrp, prev, e, ep, in_f, rf, out_f,
                 blk0, nblk):
    wide = 2 * in_f
    s = rf // out_f // _G
    yw = s * out_f

    def body(*refs):
        if prev is None:
            x_ref, s_ref, t_ref, k_ref, o_ref = refs
        else:
            x_ref, s_ref, t_ref, k_ref, _p_ref, o_ref = refs
        xw = x_ref[...]
        sv = s_ref[0]
        tv = t_ref[0]
        codev = (tv << 1) | (sv & 1)
        code = codev.reshape(_B, 1)
        slot = ((code >> 1) // s) * 2 + (code & 1)
        jrel = (code >> 1) % s
        xc = xw.astype(jnp.bfloat16)
        xg = jnp.concatenate([xc] * _G, axis=1)
        gcol = jax.lax.broadcasted_iota(jnp.int32, (_B, wide * _G), 1)
        xg = jnp.where((gcol // in_f) == slot, xg, jnp.bfloat16(0))
        y = jnp.dot(xg, k_ref[...], preferred_element_type=jnp.float32)
        jcol = jax.lax.broadcasted_iota(jnp.int32, (_B, yw), 1)
        ym = jnp.where((jcol // out_f) == jrel, y, 0.0)
        acc = ym[:, 0:128]
        for c in range(1, yw // 128):
            acc = acc + ym[:, 128 * c:128 * (c + 1)]
        res = acc[:, 0:out_f]
        for j in range(1, 128 // out_f):
            res = res + acc[:, out_f * j:out_f * (j + 1)]
        o_ref[...] = jnp.swapaxes(res, 0, 1)

    in_specs = [
        pl.BlockSpec((_B, wide), lambda i: (i, 0)),
        pl.BlockSpec((1, 1, _B), lambda i: (i + blk0, 0, 0)),
        pl.BlockSpec((1, 1, _B), lambda i: (i + blk0, 0, 0)),
        pl.BlockSpec((_G * wide, yw), lambda i: (0, 0)),
    ]
    operands = [x128, send3, type3, kgrp]
    aliases = {}
    if prev is not None:
        in_specs.append(pl.BlockSpec((out_f, _B), lambda i: (0, i + blk0)))
        operands.append(prev)
        aliases = {4: 0}

    return pl.pallas_call(
        body,
        grid=(nblk,),
        in_specs=in_specs,
        out_specs=pl.BlockSpec((out_f, _B), lambda i: (0, i + blk0)),
        out_shape=jax.ShapeDtypeStruct((out_f, e), jnp.float32),
        input_output_aliases=aliases,
    )(*operands)


def kernel(nodes, senders, edge_types, kernels):
    e = senders.shape[0]
    num_rel, in_f, out_f = kernels.shape
    rf = num_rel * out_f
    nodes2 = nodes.reshape(nodes.shape[0] // 2, 2 * in_f)

    lcm = int(np.lcm(_B, _W))
    ep = ((e + lcm - 1) // lcm) * lcm
    pad = ep - e
    sp = jnp.pad(senders, (0, pad))
    tp = jnp.pad(edge_types, (0, pad))
    nb = ep // _B

    ep2 = ep // 2
    idxh = sp >> 1
    xa = _sc_gather(nodes2, idxh[:ep2], ep2)
    xb = _sc_gather(nodes2, idxh[ep2:], ep2)

    kflat = jnp.transpose(kernels, (1, 0, 2)).reshape(in_f, rf)
    kbig = jnp.concatenate([kflat, kflat], axis=0).astype(jnp.bfloat16)
    kgrp = _regroup(kbig, _G, num_rel, out_f)

    send3 = sp.reshape(nb, 1, _B)
    type3 = tp.reshape(nb, 1, _B)
    nh = nb // 2
    outa = _tc_messages(xa, send3, type3, kgrp, None, e, ep, in_f, rf, out_f,
                        0, nh)
    outb = _tc_messages(xb, send3, type3, kgrp, outa, e, ep, in_f, rf, out_f,
                        nh, nb - nh)
    return outb.T


# speedup vs baseline: 1.3418x; 1.0903x over previous
import jax
import jax.numpy as jnp
import numpy as np
from jax.experimental import pallas as pl
from jax.experimental.pallas import tpu as pltpu
from jax.experimental.pallas import tpu_sc as plsc

_B = 2048
_W = 128
_G = 4
_SCV = 16


def _sc_gather(nodes2, idx, ep):
    feat = nodes2.shape[1]
    idx2 = idx.reshape(1, ep)
    mesh = plsc.VectorSubcoreMesh(core_axis_name="core", subcore_axis_name="subcore")

    @pl.kernel(out_type=jax.ShapeDtypeStruct((ep, feat), nodes2.dtype),
               mesh=mesh)
    def gather_kernel(x_hbm, i_hbm, o_hbm):
        def body(i_vmem, o_vmem):
            pltpu.sync_copy(x_hbm.at[i_vmem.at[0]], o_vmem)

        pltpu.emit_pipeline(
            body,
            grid=(ep // _W,),
            in_specs=[pl.BlockSpec((1, _W), index_map=lambda i: (0, i))],
            out_specs=[pl.BlockSpec((_W, feat), index_map=lambda i: (i, 0))],
            core_axis_name=("core", "subcore"),
            dimension_semantics=(pltpu.PARALLEL,),
        )(i_hbm, o_hbm)

    return gather_kernel(nodes2, idx2)


def _regroup(kbig, g, num_rel, out_f):
    wide = kbig.shape[0]
    s = num_rel // g
    return (kbig.reshape(wide, g, s * out_f)
            .transpose(1, 0, 2)
            .reshape(g * wide, s * out_f))


def _tc_messages(x128, send3, type3, kgrp, prev, e, ep, in_f, rf, out_f,
                 blk0, nblk):
    wide = 2 * in_f
    s = rf // out_f // _G
    yw = s * out_f

    def body(*refs):
        if prev is None:
            x_ref, s_ref, t_ref, k_ref, o_ref = refs
        else:
            x_ref, s_ref, t_ref, k_ref, _p_ref, o_ref = refs
        xw = x_ref[...]
        sv = s_ref[0]
        tv = t_ref[0]
        codev = (tv << 1) | (sv & 1)
        code = codev.reshape(_B, 1)
        slot = ((code >> 1) // s) * 2 + (code & 1)
        jrel = (code >> 1) % s
        xc = xw.astype(jnp.bfloat16)
        xg = jnp.concatenate([xc] * _G, axis=1)
        gcol = jax.lax.broadcasted_iota(jnp.int32, (_B, wide * _G), 1)
        xg = jnp.where((gcol // in_f) == slot, xg, jnp.bfloat16(0))
        y = jnp.dot(xg, k_ref[...], preferred_element_type=jnp.float32)
        jcol = jax.lax.broadcasted_iota(jnp.int32, (_B, yw), 1)
        ym = jnp.where((jcol // out_f) == jrel, y, 0.0)
        acc = ym[:, 0:128]
        for c in range(1, yw // 128):
            acc = acc + ym[:, 128 * c:128 * (c + 1)]
        res = acc[:, 0:out_f]
        for j in range(1, 128 // out_f):
            res = res + acc[:, out_f * j:out_f * (j + 1)]
        o_ref[...] = jnp.swapaxes(res, 0, 1)

    in_specs = [
        pl.BlockSpec((_B, wide), lambda i: (i, 0)),
        pl.BlockSpec((1, 1, _B), lambda i: (i + blk0, 0, 0)),
        pl.BlockSpec((1, 1, _B), lambda i: (i + blk0, 0, 0)),
        pl.BlockSpec((_G * wide, yw), lambda i: (0, 0)),
    ]
    operands = [x128, send3, type3, kgrp]
    aliases = {}
    if prev is not None:
        in_specs.append(pl.BlockSpec((out_f, _B), lambda i: (0, i + blk0)))
        operands.append(prev)
        aliases = {4: 0}

    return pl.pallas_call(
        body,
        grid=(nblk,),
        in_specs=in_specs,
        out_specs=pl.BlockSpec((out_f, _B), lambda i: (0, i + blk0)),
        out_shape=jax.ShapeDtypeStruct((out_f, e), jnp.float32),
        input_output_aliases=aliases,
    )(*operands)


def kernel(nodes, senders, edge_types, kernels):
    e = senders.shape[0]
    num_rel, in_f, out_f = kernels.shape
    rf = num_rel * out_f
    nodes2 = nodes.reshape(nodes.shape[0] // 2, 2 * in_f)

    lcm = int(np.lcm(_B, _W))
    ep = ((e + lcm - 1) // lcm) * lcm
    pad = ep - e
    sp = jnp.pad(senders, (0, pad))
    tp = jnp.pad(edge_types, (0, pad))
    nb = ep // _B

    idxh = sp >> 1

    kflat = jnp.transpose(kernels, (1, 0, 2)).reshape(in_f, rf)
    kbig = jnp.concatenate([kflat, kflat], axis=0).astype(jnp.bfloat16)
    kgrp = _regroup(kbig, _G, num_rel, out_f)

    send3 = sp.reshape(nb, 1, _B)
    type3 = tp.reshape(nb, 1, _B)
    nchunks = 4
    base, rem = divmod(nb, nchunks)
    sizes = [base + (1 if c < rem else 0) for c in range(nchunks)]
    out = None
    blk0 = 0
    for nblk in sizes:
        lo, hi = blk0 * _B, (blk0 + nblk) * _B
        xc = _sc_gather(nodes2, idxh[lo:hi], hi - lo)
        out = _tc_messages(xc, send3, type3, kgrp, out, e, ep, in_f, rf,
                           out_f, blk0, nblk)
        blk0 += nblk
    return out.T
